# Initial kernel scaffold; baseline (speedup 1.0000x reference)
import jax
import jax.numpy as jnp
from jax.experimental import pallas as pl


def kernel(features, edge_index_0, edge_weight_0, edge_index_1, edge_weight_1, W_proj, W1, b1, W2, b2, beta, temp, temp_list_0, temp_list_1):
    raise NotImplementedError("write your pallas kernel here")


# speedup vs baseline: 3.5619x; 3.5619x over previous
import jax
import jax.numpy as jnp
from jax import lax
from jax.experimental import pallas as pl
from jax.experimental.pallas import tpu as pltpu
from jax.experimental.pallas import tpu_sc as plsc

N = 10000
E = 160000
IN_DIM = 256
EMB_DIM = 512
H_DIM = 512
NUM_CLASSES = 8
K = 10

_BN = 1000

_NTILES = 16
_CH = 128
_CAPC = 96
_CAP = _CAPC * _CH
_RPT = 640
_NPAD = _NTILES * _RPT
_HALF = 4
_WPT = _RPT * _HALF
_AW = _WPT + 16


def _mlp_body(feat_ref, wp_ref, w1_ref, b1_ref, w2_ref, b2_ref, out_ref):
    h = feat_ref[...] @ wp_ref[...]
    h = jnp.maximum(h @ w1_ref[...] + b1_ref[...], 0.0)
    out_ref[...] = h @ w2_ref[...] + b2_ref[...]


def _mlp(features, W_proj, W1, b1, W2, b2):
    return pl.pallas_call(
        _mlp_body,
        grid=(N // _BN,),
        in_specs=[
            pl.BlockSpec((_BN, IN_DIM), lambda i: (i, 0)),
            pl.BlockSpec((IN_DIM, EMB_DIM), lambda i: (0, 0)),
            pl.BlockSpec((EMB_DIM, H_DIM), lambda i: (0, 0)),
            pl.BlockSpec((H_DIM,), lambda i: (0,)),
            pl.BlockSpec((H_DIM, NUM_CLASSES), lambda i: (0, 0)),
            pl.BlockSpec((NUM_CLASSES,), lambda i: (0,)),
        ],
        out_specs=pl.BlockSpec((_BN, NUM_CLASSES), lambda i: (i, 0)),
        out_shape=jax.ShapeDtypeStruct((N, NUM_CLASSES), jnp.float32),
    )(features, W_proj, W1, b1, W2, b2)


def _pack_edges(idx, w):
    row, col = idx[0], idx[1]
    bucket = row // _RPT
    perm = jnp.argsort(bucket, stable=True)
    row_s, col_s, val_s = row[perm], col[perm], w[perm]
    bucket_s = bucket[perm]
    counts = jnp.bincount(bucket, length=_NTILES)
    starts = jnp.concatenate([jnp.zeros((1,), counts.dtype),
                              jnp.cumsum(counts)[:-1]])
    pos = jnp.arange(E) - starts[bucket_s]
    dest = bucket_s * _CAP + pos
    rl = jnp.full((_NTILES * _CAP,), _WPT, jnp.int32)
    cl = jnp.full((_NTILES * _CAP,), N * _HALF, jnp.int32)
    vl = jnp.zeros((_NTILES * _CAP,), jnp.float32)
    rl = rl.at[dest].set((row_s - bucket_s * _RPT) * _HALF)
    cl = cl.at[dest].set(col_s * _HALF)
    vl = vl.at[dest].set(val_s)
    shape = (_NTILES, _CAPC, _CH)
    return rl.reshape(shape), cl.reshape(shape), vl.reshape(shape)


def _prop_body(x2, r0, c0, v0, r1, c1, v1, cf_hbm, res_out,
               xs, rt0, ct0, vt0, rt1, ct1, vt1, vs0, vs1,
               cibuf, gbuf, acc, res_t, resi_t, cf, sem):
    cid = lax.axis_index("c")
    sid = lax.axis_index("s")
    wbase = sid * _WPT
    iota = lax.iota(jnp.int32, 16)
    qi = iota >> 2
    li = iota & 3
    zeros16 = jnp.zeros((16,), jnp.float32)
    nvec = _WPT // 16

    def splat(i):
        return plsc.load_gather(cf, [jnp.full((16,), i, jnp.int32)])

    def vloop(n, f):
        def body(b, _):
            f(b)
            return 0
        lax.fori_loop(0, n, body, 0)

    pltpu.sync_copy(cf_hbm, cf)
    pltpu.sync_copy(r0.at[sid], rt0)
    pltpu.sync_copy(c0.at[sid], ct0)
    pltpu.sync_copy(v0.at[sid], vt0)
    pltpu.sync_copy(r1.at[sid], rt1)
    pltpu.sync_copy(c1.at[sid], ct1)
    pltpu.sync_copy(v1.at[sid], vt1)

    b0 = splat(48)
    b1 = splat(49)

    def scale_set(vs, vt, beta):
        def body(b):
            jq = jnp.full((16,), b >> 3, jnp.int32)
            a16 = (b & 7) * 16 + iota
            v = beta * plsc.load_gather(vt, [jq, a16])
            plsc.store_scatter(vs, [(b >> 3) * _CH + a16], v)
        vloop(_CAPC * 8, body)

    scale_set(vs0, vt0, b0)
    scale_set(vs1, vt1, b1)

    def zero_acc():
        vloop(_AW // 16, lambda b: acc.__setitem__(pl.ds(16 * b, 16), zeros16))

    zero_acc()
    pltpu.sync_copy(x2.at[cid, pl.ds(wbase, _WPT)], res_t)
    pltpu.sync_copy(res_t, xs.at[pl.ds(wbase, _WPT)])
    t0v = splat(0)

    def initres(b):
        sl = pl.ds(16 * b, 16)
        res_t[sl] = t0v * res_t[sl]
    vloop(nvec, initres)
    plsc.subcore_barrier()

    def run_set(rt, ct, vref):
        def chunk(j, _):
            jf = jnp.full((16,), j, jnp.int32)
            for g in range(_CH // 4):
                a16 = 4 * g + qi
                cw = plsc.load_gather(ct, [jf, a16])
                cibuf[pl.ds(16 * g, 16)] = cw + li
            pltpu.async_copy(xs.at[cibuf], gbuf, sem).wait()
            for g in range(_CH // 4):
                a16 = 4 * g + qi
                vv = vref(jf, a16, j)
                rw = plsc.load_gather(rt, [jf, a16])
                m = vv * gbuf[pl.ds(16 * g, 16)]
                plsc.addupdate_scatter(acc, [rw + li], m)
            return 0
        lax.fori_loop(0, _CAPC, chunk, 0)

    def v_from_2d(vt):
        return lambda jf, a16, j: plsc.load_gather(vt, [jf, a16])

    def v_from_1d(vs):
        return lambda jf, a16, j: plsc.load_gather(vs, [j * _CH + a16])

    def iteration(k, _):
        @pl.when(k < 10)
        def _():
            run_set(rt0, ct0, v_from_1d(vs0))
            run_set(rt1, ct1, v_from_1d(vs1))

        @pl.when(jnp.logical_and(k >= 10, k < 20))
        def _():
            run_set(rt0, ct0, v_from_2d(vt0))

        @pl.when(k >= 20)
        def _():
            run_set(rt1, ct1, v_from_2d(vt1))

        ci = jnp.where(k < 10, k + 1, jnp.where(k < 20, k + 7, k + 13))
        dk = plsc.load_gather(cf, [jnp.full((16,), ci, jnp.int32)])

        @pl.when(k == 9)
        def _():
            t = splat(16)

            def body(b):
                sl = pl.ds(16 * b, 16)
                resi_t[sl] = t * acc[sl]
            vloop(nvec, body)

        def fold(dst):
            def body(b):
                sl = pl.ds(16 * b, 16)
                dst[sl] = dst[sl] + dk * acc[sl]
            vloop(nvec, body)

        @pl.when(k < 10)
        def _():
            fold(res_t)

        @pl.when(k >= 10)
        def _():
            fold(resi_t)

        @pl.when(k == 19)
        def _():
            t = splat(32)

            def body(b):
                sl = pl.ds(16 * b, 16)
                res_t[sl] = res_t[sl] + resi_t[sl]
                resi_t[sl] = t * acc[sl]
            vloop(nvec, body)

        @pl.when(k == 29)
        def _():
            def body(b):
                sl = pl.ds(16 * b, 16)
                res_t[sl] = res_t[sl] + resi_t[sl]
            vloop(nvec, body)

        plsc.subcore_barrier()
        pltpu.sync_copy(acc.at[pl.ds(0, _WPT)], xs.at[pl.ds(wbase, _WPT)])
        zero_acc()
        plsc.subcore_barrier()
        return 0

    lax.fori_loop(0, 3 * K, iteration, 0)
    pltpu.sync_copy(res_t, res_out.at[cid, pl.ds(wbase, _WPT)])


_PROP_OUT = jax.ShapeDtypeStruct((2, _NPAD * _HALF), jnp.float32)
_PROP_SCRATCH = [
    pltpu.VMEM_SHARED((_NPAD * _HALF,), jnp.float32),
    pltpu.VMEM((_CAPC, _CH), jnp.int32),
    pltpu.VMEM((_CAPC, _CH), jnp.int32),
    pltpu.VMEM((_CAPC, _CH), jnp.float32),
    pltpu.VMEM((_CAPC, _CH), jnp.int32),
    pltpu.VMEM((_CAPC, _CH), jnp.int32),
    pltpu.VMEM((_CAPC, _CH), jnp.float32),
    pltpu.VMEM((_CAP,), jnp.float32),
    pltpu.VMEM((_CAP,), jnp.float32),
    pltpu.VMEM((_CH * _HALF,), jnp.int32),
    pltpu.VMEM((_CH * _HALF,), jnp.float32),
    pltpu.VMEM((_AW,), jnp.float32),
    pltpu.VMEM((_WPT,), jnp.float32),
    pltpu.VMEM((_WPT,), jnp.float32),
    pltpu.VMEM((64,), jnp.float32),
    pltpu.SemaphoreType.DMA,
]

_prop = pl.kernel(
    _prop_body,
    out_type=_PROP_OUT,
    mesh=plsc.VectorSubcoreMesh(core_axis_name="c", subcore_axis_name="s"),
    scratch_types=_PROP_SCRATCH,
    compiler_params=pltpu.CompilerParams(needs_layout_passes=False),
)


def kernel(features, edge_index_0, edge_weight_0, edge_index_1, edge_weight_1,
           W_proj, W1, b1, W2, b2, beta, temp, temp_list_0, temp_list_1):
    x = _mlp(features, W_proj, W1, b1, W2, b2)

    x2 = jnp.pad(x, ((0, _NPAD - N), (0, 0)))
    x2 = x2.reshape(_NPAD, 2, _HALF).transpose(1, 0, 2).reshape(2, _NPAD * _HALF)
    r0, c0, v0 = _pack_edges(edge_index_0, edge_weight_0)
    r1, c1, v1 = _pack_edges(edge_index_1, edge_weight_1)

    beta_s = jax.nn.softmax(beta, axis=0)
    cf = jnp.zeros((64,), jnp.float32)
    cf = cf.at[0:11].set(temp)
    cf = cf.at[16:27].set(temp_list_0)
    cf = cf.at[32:43].set(temp_list_1)
    cf = cf.at[48].set(beta_s[0, 0])
    cf = cf.at[49].set(beta_s[1, 0])

    res2 = _prop(x2, r0, c0, v0, r1, c1, v1, cf)
    return res2.reshape(2, _NPAD, _HALF).transpose(1, 0, 2).reshape(_NPAD, 8)[:N]


# speedup vs baseline: 6.1818x; 1.7355x over previous
import jax
import jax.numpy as jnp
from jax import lax
from jax.experimental import pallas as pl
from jax.experimental.pallas import tpu as pltpu
from jax.experimental.pallas import tpu_sc as plsc

N = 10000
E = 160000
IN_DIM = 256
EMB_DIM = 512
H_DIM = 512
NUM_CLASSES = 8
K = 10

_BN = 1000

_NTILES = 16
_CH = 128
_CAPC = 96
_CAP = _CAPC * _CH
_RPT = 640
_NPAD = _NTILES * _RPT
_HALF = 4
_WPT = _RPT * _HALF
_AW = _WPT + 16


def _mlp_body(feat_ref, wp_ref, w1_ref, b1_ref, w2_ref, b2_ref, out_ref):
    h = feat_ref[...] @ wp_ref[...]
    h = jnp.maximum(h @ w1_ref[...] + b1_ref[...], 0.0)
    out_ref[...] = h @ w2_ref[...] + b2_ref[...]


def _mlp(features, W_proj, W1, b1, W2, b2):
    return pl.pallas_call(
        _mlp_body,
        grid=(N // _BN,),
        in_specs=[
            pl.BlockSpec((_BN, IN_DIM), lambda i: (i, 0)),
            pl.BlockSpec((IN_DIM, EMB_DIM), lambda i: (0, 0)),
            pl.BlockSpec((EMB_DIM, H_DIM), lambda i: (0, 0)),
            pl.BlockSpec((H_DIM,), lambda i: (0,)),
            pl.BlockSpec((H_DIM, NUM_CLASSES), lambda i: (0, 0)),
            pl.BlockSpec((NUM_CLASSES,), lambda i: (0,)),
        ],
        out_specs=pl.BlockSpec((_BN, NUM_CLASSES), lambda i: (i, 0)),
        out_shape=jax.ShapeDtypeStruct((N, NUM_CLASSES), jnp.float32),
    )(features, W_proj, W1, b1, W2, b2)


def _pack_edges(idx, w):
    row, col = idx[0], idx[1]
    bucket = row // _RPT
    perm = jnp.argsort(bucket, stable=True)
    row_s, col_s, val_s = row[perm], col[perm], w[perm]
    bucket_s = bucket[perm]
    bounds = jnp.searchsorted(bucket_s, jnp.arange(_NTILES + 1, dtype=jnp.int32))
    starts, ends = bounds[:-1], bounds[1:]
    pos = jnp.arange(_CAP, dtype=jnp.int32)[None, :]
    src = jnp.minimum(starts[:, None] + pos, E - 1)
    valid = pos < (ends - starts)[:, None]
    tb = jnp.arange(_NTILES, dtype=jnp.int32)[:, None]
    rl = jnp.where(valid, (row_s[src] - tb * _RPT) * _HALF, _WPT)
    cl = jnp.where(valid, col_s[src] * _HALF, N * _HALF)
    vl = jnp.where(valid, val_s[src], jnp.float32(0.0))
    shape = (_NTILES, _CAPC, _CH)
    return (rl.astype(jnp.int32).reshape(shape),
            cl.astype(jnp.int32).reshape(shape),
            vl.reshape(shape))


def _prop_body(x2, r0, c0, v0, r1, c1, v1, cf_hbm, res_out,
               xs, rt0, ct0, vt0, rt1, ct1, vt1, vs0, vs1,
               cibuf, gbuf, acc, res_t, resi_t, cf, sem):
    cid = lax.axis_index("c")
    sid = lax.axis_index("s")
    wbase = sid * _WPT
    iota = lax.iota(jnp.int32, 16)
    qi = iota >> 2
    li = iota & 3
    zeros16 = jnp.zeros((16,), jnp.float32)
    nvec = _WPT // 16

    def splat(i):
        return plsc.load_gather(cf, [jnp.full((16,), i, jnp.int32)])

    def vloop(n, f):
        def body(b, _):
            f(b)
            return 0
        lax.fori_loop(0, n, body, 0)

    pltpu.sync_copy(cf_hbm, cf)
    pltpu.sync_copy(r0.at[sid], rt0)
    pltpu.sync_copy(c0.at[sid], ct0)
    pltpu.sync_copy(v0.at[sid], vt0)
    pltpu.sync_copy(r1.at[sid], rt1)
    pltpu.sync_copy(c1.at[sid], ct1)
    pltpu.sync_copy(v1.at[sid], vt1)

    b0 = splat(48)
    b1 = splat(49)

    def scale_set(vs, vt, beta):
        def body(b):
            jq = jnp.full((16,), b >> 3, jnp.int32)
            a16 = (b & 7) * 16 + iota
            v = beta * plsc.load_gather(vt, [jq, a16])
            plsc.store_scatter(vs, [(b >> 3) * _CH + a16], v)
        vloop(_CAPC * 8, body)

    scale_set(vs0, vt0, b0)
    scale_set(vs1, vt1, b1)

    def zero_acc():
        vloop(_AW // 16, lambda b: acc.__setitem__(pl.ds(16 * b, 16), zeros16))

    zero_acc()
    pltpu.sync_copy(x2.at[cid, pl.ds(wbase, _WPT)], res_t)
    pltpu.sync_copy(res_t, xs.at[pl.ds(wbase, _WPT)])
    t0v = splat(0)

    def initres(b):
        sl = pl.ds(16 * b, 16)
        res_t[sl] = t0v * res_t[sl]
    vloop(nvec, initres)
    plsc.subcore_barrier()

    def run_set(rt, ct, vref):
        def chunk(j, _):
            jf = jnp.full((16,), j, jnp.int32)
            for g in range(_CH // 4):
                a16 = 4 * g + qi
                cw = plsc.load_gather(ct, [jf, a16])
                cibuf[pl.ds(16 * g, 16)] = cw + li
            pltpu.async_copy(xs.at[cibuf], gbuf, sem).wait()
            for g in range(_CH // 4):
                a16 = 4 * g + qi
                vv = vref(jf, a16, j)
                rw = plsc.load_gather(rt, [jf, a16])
                m = vv * gbuf[pl.ds(16 * g, 16)]
                plsc.addupdate_scatter(acc, [rw + li], m)
            return 0
        lax.fori_loop(0, _CAPC, chunk, 0)

    def v_from_2d(vt):
        return lambda jf, a16, j: plsc.load_gather(vt, [jf, a16])

    def v_from_1d(vs):
        return lambda jf, a16, j: plsc.load_gather(vs, [j * _CH + a16])

    def iteration(k, _):
        @pl.when(k < 10)
        def _():
            run_set(rt0, ct0, v_from_1d(vs0))
            run_set(rt1, ct1, v_from_1d(vs1))

        @pl.when(jnp.logical_and(k >= 10, k < 20))
        def _():
            run_set(rt0, ct0, v_from_2d(vt0))

        @pl.when(k >= 20)
        def _():
            run_set(rt1, ct1, v_from_2d(vt1))

        ci = jnp.where(k < 10, k + 1, jnp.where(k < 20, k + 7, k + 13))
        dk = plsc.load_gather(cf, [jnp.full((16,), ci, jnp.int32)])

        @pl.when(k == 9)
        def _():
            t = splat(16)

            def body(b):
                sl = pl.ds(16 * b, 16)
                resi_t[sl] = t * acc[sl]
            vloop(nvec, body)

        def fold(dst):
            def body(b):
                sl = pl.ds(16 * b, 16)
                dst[sl] = dst[sl] + dk * acc[sl]
            vloop(nvec, body)

        @pl.when(k < 10)
        def _():
            fold(res_t)

        @pl.when(k >= 10)
        def _():
            fold(resi_t)

        @pl.when(k == 19)
        def _():
            t = splat(32)

            def body(b):
                sl = pl.ds(16 * b, 16)
                res_t[sl] = res_t[sl] + resi_t[sl]
                resi_t[sl] = t * acc[sl]
            vloop(nvec, body)

        @pl.when(k == 29)
        def _():
            def body(b):
                sl = pl.ds(16 * b, 16)
                res_t[sl] = res_t[sl] + resi_t[sl]
            vloop(nvec, body)

        plsc.subcore_barrier()
        pltpu.sync_copy(acc.at[pl.ds(0, _WPT)], xs.at[pl.ds(wbase, _WPT)])
        zero_acc()
        plsc.subcore_barrier()
        return 0

    lax.fori_loop(0, 3 * K, iteration, 0)
    pltpu.sync_copy(res_t, res_out.at[cid, pl.ds(wbase, _WPT)])


_PROP_OUT = jax.ShapeDtypeStruct((2, _NPAD * _HALF), jnp.float32)
_PROP_SCRATCH = [
    pltpu.VMEM_SHARED((_NPAD * _HALF,), jnp.float32),
    pltpu.VMEM((_CAPC, _CH), jnp.int32),
    pltpu.VMEM((_CAPC, _CH), jnp.int32),
    pltpu.VMEM((_CAPC, _CH), jnp.float32),
    pltpu.VMEM((_CAPC, _CH), jnp.int32),
    pltpu.VMEM((_CAPC, _CH), jnp.int32),
    pltpu.VMEM((_CAPC, _CH), jnp.float32),
    pltpu.VMEM((_CAP,), jnp.float32),
    pltpu.VMEM((_CAP,), jnp.float32),
    pltpu.VMEM((_CH * _HALF,), jnp.int32),
    pltpu.VMEM((_CH * _HALF,), jnp.float32),
    pltpu.VMEM((_AW,), jnp.float32),
    pltpu.VMEM((_WPT,), jnp.float32),
    pltpu.VMEM((_WPT,), jnp.float32),
    pltpu.VMEM((64,), jnp.float32),
    pltpu.SemaphoreType.DMA,
]

_prop = pl.kernel(
    _prop_body,
    out_type=_PROP_OUT,
    mesh=plsc.VectorSubcoreMesh(core_axis_name="c", subcore_axis_name="s"),
    scratch_types=_PROP_SCRATCH,
    compiler_params=pltpu.CompilerParams(needs_layout_passes=False),
)


def kernel(features, edge_index_0, edge_weight_0, edge_index_1, edge_weight_1,
           W_proj, W1, b1, W2, b2, beta, temp, temp_list_0, temp_list_1):
    x = _mlp(features, W_proj, W1, b1, W2, b2)

    x2 = jnp.pad(x, ((0, _NPAD - N), (0, 0)))
    x2 = x2.reshape(_NPAD, 2, _HALF).transpose(1, 0, 2).reshape(2, _NPAD * _HALF)
    r0, c0, v0 = _pack_edges(edge_index_0, edge_weight_0)
    r1, c1, v1 = _pack_edges(edge_index_1, edge_weight_1)

    beta_s = jax.nn.softmax(beta, axis=0)
    cf = jnp.zeros((64,), jnp.float32)
    cf = cf.at[0:11].set(temp)
    cf = cf.at[16:27].set(temp_list_0)
    cf = cf.at[32:43].set(temp_list_1)
    cf = cf.at[48].set(beta_s[0, 0])
    cf = cf.at[49].set(beta_s[1, 0])

    res2 = _prop(x2, r0, c0, v0, r1, c1, v1, cf)
    return res2.reshape(2, _NPAD, _HALF).transpose(1, 0, 2).reshape(_NPAD, 8)[:N]


# speedup vs baseline: 7.4761x; 1.2094x over previous
import jax
import jax.numpy as jnp
from jax import lax
from jax.experimental import pallas as pl
from jax.experimental.pallas import tpu as pltpu
from jax.experimental.pallas import tpu_sc as plsc

N = 10000
E = 160000
IN_DIM = 256
EMB_DIM = 512
H_DIM = 512
NUM_CLASSES = 8
K = 10

_BN = 1000

_NTILES = 16
_CH = 128
_CAPC = 96
_CAP = _CAPC * _CH
_RPT = 640
_NPAD = _NTILES * _RPT
_HALF = 4
_WPT = _RPT * _HALF
_AW = _WPT + 16


def _mlp_body(feat_ref, wp_ref, w1_ref, b1_ref, w2_ref, b2_ref, out_ref):
    h = feat_ref[...] @ wp_ref[...]
    h = jnp.maximum(h @ w1_ref[...] + b1_ref[...], 0.0)
    out_ref[...] = h @ w2_ref[...] + b2_ref[...]


def _mlp(features, W_proj, W1, b1, W2, b2):
    return pl.pallas_call(
        _mlp_body,
        grid=(N // _BN,),
        in_specs=[
            pl.BlockSpec((_BN, IN_DIM), lambda i: (i, 0)),
            pl.BlockSpec((IN_DIM, EMB_DIM), lambda i: (0, 0)),
            pl.BlockSpec((EMB_DIM, H_DIM), lambda i: (0, 0)),
            pl.BlockSpec((H_DIM,), lambda i: (0,)),
            pl.BlockSpec((H_DIM, NUM_CLASSES), lambda i: (0, 0)),
            pl.BlockSpec((NUM_CLASSES,), lambda i: (0,)),
        ],
        out_specs=pl.BlockSpec((_BN, NUM_CLASSES), lambda i: (i, 0)),
        out_shape=jax.ShapeDtypeStruct((N, NUM_CLASSES), jnp.float32),
    )(features, W_proj, W1, b1, W2, b2)


def _pack_edges(idx, w):
    row, col = idx[0], idx[1]
    bucket = row // _RPT
    perm = jnp.argsort(bucket, stable=True)
    row_s, col_s, val_s = row[perm], col[perm], w[perm]
    bucket_s = bucket[perm]
    bounds = jnp.searchsorted(bucket_s, jnp.arange(_NTILES + 1, dtype=jnp.int32))
    starts, ends = bounds[:-1], bounds[1:]
    pos = jnp.arange(_CAP, dtype=jnp.int32)[None, :]
    src = jnp.minimum(starts[:, None] + pos, E - 1)
    valid = pos < (ends - starts)[:, None]
    tb = jnp.arange(_NTILES, dtype=jnp.int32)[:, None]
    rl = jnp.where(valid, (row_s[src] - tb * _RPT) * _HALF, _WPT)
    cl = jnp.where(valid, col_s[src] * _HALF, N * _HALF)
    vl = jnp.where(valid, val_s[src], jnp.float32(0.0))
    shape = (_NTILES, _CAPC, _CH)
    return (rl.astype(jnp.int32).reshape(shape),
            cl.astype(jnp.int32).reshape(shape),
            vl.reshape(shape))


def _prop_body(x2, r0, c0, v0, r1, c1, v1, cf_hbm, res_out,
               xs, rt0, ct0, vt0, rt1, ct1, vt1, vs0, vs1,
               cibuf, gbuf, cibuf2, gbuf2, acc, res_t, resi_t, cf, sem, sem2):
    cid = lax.axis_index("c")
    sid = lax.axis_index("s")
    wbase = sid * _WPT
    iota = lax.iota(jnp.int32, 16)
    qi = iota >> 2
    li = iota & 3
    zeros16 = jnp.zeros((16,), jnp.float32)
    nvec = _WPT // 16

    def splat(i):
        return plsc.load_gather(cf, [jnp.full((16,), i, jnp.int32)])

    def vloop(n, f):
        def body(b, _):
            f(b)
            return 0
        lax.fori_loop(0, n, body, 0)

    pltpu.sync_copy(cf_hbm, cf)
    pltpu.sync_copy(r0.at[sid], rt0)
    pltpu.sync_copy(c0.at[sid], ct0)
    pltpu.sync_copy(v0.at[sid], vt0)
    pltpu.sync_copy(r1.at[sid], rt1)
    pltpu.sync_copy(c1.at[sid], ct1)
    pltpu.sync_copy(v1.at[sid], vt1)

    b0 = splat(48)
    b1 = splat(49)

    def scale_set(vs, vt, beta):
        def body(b):
            jq = jnp.full((16,), b >> 3, jnp.int32)
            a16 = (b & 7) * 16 + iota
            v = beta * plsc.load_gather(vt, [jq, a16])
            plsc.store_scatter(vs, [(b >> 3) * _CH + a16], v)
        vloop(_CAPC * 8, body)

    scale_set(vs0, vt0, b0)
    scale_set(vs1, vt1, b1)

    def zero_acc():
        vloop(_AW // 16, lambda b: acc.__setitem__(pl.ds(16 * b, 16), zeros16))

    zero_acc()
    pltpu.sync_copy(x2.at[cid, pl.ds(wbase, _WPT)], res_t)
    pltpu.sync_copy(res_t, xs.at[pl.ds(wbase, _WPT)])
    t0v = splat(0)

    def initres(b):
        sl = pl.ds(16 * b, 16)
        res_t[sl] = t0v * res_t[sl]
    vloop(nvec, initres)
    plsc.subcore_barrier()

    def run_set(rt, ct, vref):
        def expand(j, cib):
            jf = jnp.full((16,), j, jnp.int32)
            for g in range(_CH // 4):
                a16 = 4 * g + qi
                cw = plsc.load_gather(ct, [jf, a16])
                cib[pl.ds(16 * g, 16)] = cw + li

        def compute(j, gb):
            jf = jnp.full((16,), j, jnp.int32)
            for g in range(_CH // 4):
                a16 = 4 * g + qi
                vv = vref(jf, a16, j)
                rw = plsc.load_gather(rt, [jf, a16])
                m = vv * gb[pl.ds(16 * g, 16)]
                plsc.addupdate_scatter(acc, [rw + li], m)

        expand(0, cibuf)
        pltpu.async_copy(xs.at[cibuf], gbuf, sem)

        def body(i, _):
            j0 = 2 * i
            expand(j0 + 1, cibuf2)
            pltpu.async_copy(xs.at[cibuf2], gbuf2, sem2)
            pltpu.make_async_copy(xs.at[cibuf], gbuf, sem).wait()
            compute(j0, gbuf)

            @pl.when(j0 + 2 < _CAPC)
            def _():
                expand(j0 + 2, cibuf)
                pltpu.async_copy(xs.at[cibuf], gbuf, sem)
            pltpu.make_async_copy(xs.at[cibuf2], gbuf2, sem2).wait()
            compute(j0 + 1, gbuf2)
            return 0
        lax.fori_loop(0, _CAPC // 2, body, 0)

    def v_from_2d(vt):
        return lambda jf, a16, j: plsc.load_gather(vt, [jf, a16])

    def v_from_1d(vs):
        return lambda jf, a16, j: plsc.load_gather(vs, [j * _CH + a16])

    def iteration(k, _):
        @pl.when(k < 10)
        def _():
            run_set(rt0, ct0, v_from_1d(vs0))
            run_set(rt1, ct1, v_from_1d(vs1))

        @pl.when(jnp.logical_and(k >= 10, k < 20))
        def _():
            run_set(rt0, ct0, v_from_2d(vt0))

        @pl.when(k >= 20)
        def _():
            run_set(rt1, ct1, v_from_2d(vt1))

        ci = jnp.where(k < 10, k + 1, jnp.where(k < 20, k + 7, k + 13))
        dk = plsc.load_gather(cf, [jnp.full((16,), ci, jnp.int32)])

        @pl.when(k == 9)
        def _():
            t = splat(16)

            def body(b):
                sl = pl.ds(16 * b, 16)
                resi_t[sl] = t * acc[sl]
            vloop(nvec, body)

        def fold(dst):
            def body(b):
                sl = pl.ds(16 * b, 16)
                dst[sl] = dst[sl] + dk * acc[sl]
            vloop(nvec, body)

        @pl.when(k < 10)
        def _():
            fold(res_t)

        @pl.when(k >= 10)
        def _():
            fold(resi_t)

        @pl.when(k == 19)
        def _():
            t = splat(32)

            def body(b):
                sl = pl.ds(16 * b, 16)
                res_t[sl] = res_t[sl] + resi_t[sl]
                resi_t[sl] = t * acc[sl]
            vloop(nvec, body)

        @pl.when(k == 29)
        def _():
            def body(b):
                sl = pl.ds(16 * b, 16)
                res_t[sl] = res_t[sl] + resi_t[sl]
            vloop(nvec, body)

        plsc.subcore_barrier()
        pltpu.sync_copy(acc.at[pl.ds(0, _WPT)], xs.at[pl.ds(wbase, _WPT)])
        zero_acc()
        plsc.subcore_barrier()
        return 0

    lax.fori_loop(0, 3 * K, iteration, 0)
    pltpu.sync_copy(res_t, res_out.at[cid, pl.ds(wbase, _WPT)])


_PROP_OUT = jax.ShapeDtypeStruct((2, _NPAD * _HALF), jnp.float32)
_PROP_SCRATCH = [
    pltpu.VMEM_SHARED((_NPAD * _HALF,), jnp.float32),
    pltpu.VMEM((_CAPC, _CH), jnp.int32),
    pltpu.VMEM((_CAPC, _CH), jnp.int32),
    pltpu.VMEM((_CAPC, _CH), jnp.float32),
    pltpu.VMEM((_CAPC, _CH), jnp.int32),
    pltpu.VMEM((_CAPC, _CH), jnp.int32),
    pltpu.VMEM((_CAPC, _CH), jnp.float32),
    pltpu.VMEM((_CAP,), jnp.float32),
    pltpu.VMEM((_CAP,), jnp.float32),
    pltpu.VMEM((_CH * _HALF,), jnp.int32),
    pltpu.VMEM((_CH * _HALF,), jnp.float32),
    pltpu.VMEM((_CH * _HALF,), jnp.int32),
    pltpu.VMEM((_CH * _HALF,), jnp.float32),
    pltpu.VMEM((_AW,), jnp.float32),
    pltpu.VMEM((_WPT,), jnp.float32),
    pltpu.VMEM((_WPT,), jnp.float32),
    pltpu.VMEM((64,), jnp.float32),
    pltpu.SemaphoreType.DMA,
    pltpu.SemaphoreType.DMA,
]

_prop = pl.kernel(
    _prop_body,
    out_type=_PROP_OUT,
    mesh=plsc.VectorSubcoreMesh(core_axis_name="c", subcore_axis_name="s"),
    scratch_types=_PROP_SCRATCH,
    compiler_params=pltpu.CompilerParams(needs_layout_passes=False),
)


def kernel(features, edge_index_0, edge_weight_0, edge_index_1, edge_weight_1,
           W_proj, W1, b1, W2, b2, beta, temp, temp_list_0, temp_list_1):
    x = _mlp(features, W_proj, W1, b1, W2, b2)

    x2 = jnp.pad(x, ((0, _NPAD - N), (0, 0)))
    x2 = x2.reshape(_NPAD, 2, _HALF).transpose(1, 0, 2).reshape(2, _NPAD * _HALF)
    r0, c0, v0 = _pack_edges(edge_index_0, edge_weight_0)
    r1, c1, v1 = _pack_edges(edge_index_1, edge_weight_1)

    beta_s = jax.nn.softmax(beta, axis=0)
    cf = jnp.zeros((64,), jnp.float32)
    cf = cf.at[0:11].set(temp)
    cf = cf.at[16:27].set(temp_list_0)
    cf = cf.at[32:43].set(temp_list_1)
    cf = cf.at[48].set(beta_s[0, 0])
    cf = cf.at[49].set(beta_s[1, 0])

    res2 = _prop(x2, r0, c0, v0, r1, c1, v1, cf)
    return res2.reshape(2, _NPAD, _HALF).transpose(1, 0, 2).reshape(_NPAD, 8)[:N]
